# Initial kernel scaffold; baseline (speedup 1.0000x reference)
import functools

import jax
import jax.numpy as jnp
from jax import lax
from jax.experimental import pallas as pl
from jax.experimental.pallas import tpu as pltpu
from jax.experimental.pallas import tpu_sc as plsc

NUM_REL = 8
NC = 2
NS = 16
LANE = 16
G = 128
ZR = 125


def _make_agg(N, E, D, want_cnt):
    rel_per_core = NUM_REL // NC
    share = E // NS
    sb = 2000
    nb = share // sb
    cap = share + G
    ng = (share + G - 1) // G
    slab_rows = N + LANE
    trash = N
    stripe = N // NS

    assert share % sb == 0 and N % NS == 0 and stripe % ZR == 0 and sb % LANE == 0

    mesh = plsc.VectorSubcoreMesh(core_axis_name="c", subcore_axis_name="s")

    out_type = [jax.ShapeDtypeStruct((NUM_REL, N, D), jnp.float32)]
    if want_cnt:
        out_type.append(jax.ShapeDtypeStruct((NUM_REL, N, LANE), jnp.float32))

    scratch = [
        pltpu.VMEM((2, sb), jnp.int32),
        pltpu.VMEM((2, sb), jnp.int32),
        pltpu.VMEM((2, sb), jnp.int32),
        pltpu.VMEM((cap,), jnp.int32),
        pltpu.VMEM((cap,), jnp.int32),
        pltpu.VMEM((G, D), jnp.float32),
        pltpu.VMEM((G,), jnp.int32),
        pltpu.VMEM((ZR, D), jnp.float32),
        pltpu.VMEM((ZR, LANE), jnp.float32),
        pltpu.VMEM((G, LANE), jnp.float32),
        pltpu.VMEM_SHARED((slab_rows, D), jnp.float32),
        pltpu.VMEM_SHARED((slab_rows, LANE), jnp.float32),
        pltpu.SemaphoreType.DMA,
        pltpu.SemaphoreType.DMA,
        pltpu.SemaphoreType.DMA,
    ]

    @functools.partial(pl.kernel, out_type=out_type, mesh=mesh,
                       scratch_types=scratch)
    def agg_kernel(x_hbm, src_hbm, dst_hbm, typ_hbm, zrow_hbm, zrow16_hbm,
                   ones_hbm, agg_hbm, *rest):
        if want_cnt:
            cnt_hbm = rest[0]
            rest = rest[1:]
        (ebuf_s, ebuf_d, ebuf_t, match_src, match_dst, rows, idx_stage,
         zbuf, zbuf16, ones_v, slab, cnt_slab, esem0, esem1, gsem) = rest

        c = lax.axis_index("c")
        s = lax.axis_index("s")
        s_base = s * share
        row0 = s * stripe

        pltpu.sync_copy(zrow_hbm, zbuf)
        pltpu.sync_copy(zrow16_hbm, zbuf16)
        if want_cnt:
            pltpu.sync_copy(ones_hbm, ones_v)

        esems = (esem0, esem1)

        def fire(b, slot):
            base = s_base + b * sb
            sem = esems[slot]
            return (
                pltpu.async_copy(src_hbm.at[pl.ds(base, sb)], ebuf_s.at[slot], sem),
                pltpu.async_copy(dst_hbm.at[pl.ds(base, sb)], ebuf_d.at[slot], sem),
                pltpu.async_copy(typ_hbm.at[pl.ds(base, sb)], ebuf_t.at[slot], sem),
            )

        pad_src = jnp.zeros((LANE,), jnp.int32)
        pad_dst = jnp.full((LANE,), trash, jnp.int32)

        for rr in range(rel_per_core):
            r_idx = c * rel_per_core + rr

            for z in range(stripe // ZR):
                pltpu.sync_copy(zbuf, slab.at[pl.ds(row0 + z * ZR, ZR)])
                if want_cnt:
                    pltpu.sync_copy(zbuf16, cnt_slab.at[pl.ds(row0 + z * ZR, ZR)])
            plsc.subcore_barrier()

            def make_sbody(slot):
                def sbody(i, m):
                    off = i * LANE
                    sv = ebuf_s[slot, pl.ds(off, LANE)]
                    dv = ebuf_d[slot, pl.ds(off, LANE)]
                    tv = ebuf_t[slot, pl.ds(off, LANE)]
                    mask = tv == r_idx
                    plsc.store_compressed(match_src.at[pl.ds(m, LANE)], sv, mask=mask)
                    plsc.store_compressed(match_dst.at[pl.ds(m, LANE)], dv, mask=mask)
                    return m + jnp.sum(mask.astype(jnp.int32))
                return sbody

            descs = {0: fire(0, 0)}
            m = jnp.int32(0)
            for b in range(nb):
                slot = b % 2
                if b + 1 < nb:
                    descs[b + 1] = fire(b + 1, 1 - slot)
                for dd in descs.pop(b):
                    dd.wait()
                m = lax.fori_loop(0, sb // LANE, make_sbody(slot), m)

            for t in range(G // LANE):
                match_src[pl.ds(m + t * LANE, LANE)] = pad_src
                match_dst[pl.ds(m + t * LANE, LANE)] = pad_dst

            def gbody(g, carry):
                @pl.when(g * G < m)
                def _():
                    idx = match_src.at[pl.ds(g * G, G)]
                    pltpu.async_copy(x_hbm.at[idx], rows, gsem).wait()
                    for t in range(G // LANE):
                        idx_stage[pl.ds(t * LANE, LANE)] = (
                            match_dst[pl.ds(g * G + t * LANE, LANE)])
                    pltpu.sync_copy(rows, slab.at[idx_stage], add=True)
                    if want_cnt:
                        pltpu.sync_copy(ones_v, cnt_slab.at[idx_stage], add=True)
                return carry
            lax.fori_loop(0, ng, gbody, jnp.int32(0))

            plsc.subcore_barrier()

            pltpu.sync_copy(slab.at[pl.ds(row0, stripe)],
                            agg_hbm.at[r_idx, pl.ds(row0, stripe)])
            if want_cnt:
                pltpu.sync_copy(cnt_slab.at[pl.ds(row0, stripe)],
                                cnt_hbm.at[r_idx, pl.ds(row0, stripe)])

    return agg_kernel


def _dense_body(relu, x_ref, agg_ref, cnt_ref, w_ref, root_ref, b_ref, o_ref):
    acc = jnp.dot(x_ref[...], root_ref[...],
                  preferred_element_type=jnp.float32) + b_ref[...]
    for rr in range(NUM_REL):
        a = agg_ref[rr]
        cnt = cnt_ref[rr, :, 0:1]
        a = a / jnp.maximum(cnt, 1.0)
        acc = acc + jnp.dot(a, w_ref[rr], preferred_element_type=jnp.float32)
    if relu:
        acc = jnp.maximum(acc, 0.0)
    o_ref[...] = acc


def _dense(x, agg, cnt, W, root, b, relu):
    N, D = x.shape
    blk = 256
    nblk = (N + blk - 1) // blk
    return pl.pallas_call(
        functools.partial(_dense_body, relu),
        grid=(nblk,),
        in_specs=[
            pl.BlockSpec((blk, D), lambda i: (i, 0)),
            pl.BlockSpec((NUM_REL, blk, D), lambda i: (0, i, 0)),
            pl.BlockSpec((NUM_REL, blk, LANE), lambda i: (0, i, 0)),
            pl.BlockSpec((NUM_REL, D, D), lambda i: (0, 0, 0)),
            pl.BlockSpec((D, D), lambda i: (0, 0)),
            pl.BlockSpec((1, D), lambda i: (0, 0)),
        ],
        out_specs=pl.BlockSpec((blk, D), lambda i: (i, 0)),
        out_shape=jax.ShapeDtypeStruct((N, D), jnp.float32),
    )(x, agg, cnt, W, root, b.reshape(1, D))


def kernel(x, edge_indexes, edge_types, W1, root1, b1, W2, root2, b2):
    N, D = x.shape
    E = edge_types.shape[0]
    src = edge_indexes[0]
    dst = edge_indexes[1]

    zrow = jnp.zeros((ZR, D), jnp.float32)
    zrow16 = jnp.zeros((ZR, LANE), jnp.float32)
    ones = jnp.ones((G, LANE), jnp.float32)

    agg_cnt = _make_agg(N, E, D, True)
    agg_only = _make_agg(N, E, D, False)

    agg1, cnt = agg_cnt(x, src, dst, edge_types, zrow, zrow16, ones)
    h = _dense(x, agg1, cnt, W1, root1, b1, True)
    agg2 = agg_only(h, src, dst, edge_types, zrow, zrow16, ones)
    out = _dense(h, agg2, cnt, W2, root2, b2, False)
    return out


# speedup vs baseline: 3.9438x; 3.9438x over previous
import functools

import jax
import jax.numpy as jnp
from jax import lax
from jax.experimental import pallas as pl
from jax.experimental.pallas import tpu as pltpu
from jax.experimental.pallas import tpu_sc as plsc

NUM_REL = 8
NC = 2
NS = 16
LANE = 16
G = 128


def _make_agg(N, E_pad, D, want_cnt):
    rel_per_core = NUM_REL // NC
    share = E_pad // NS
    sb = 2048
    nb = share // sb
    cap = share + G
    ng = (share + G - 1) // G
    nq = 4
    half = -(-(-(-N // nq)) // (8 * NS)) * (8 * NS)
    npad = nq * half
    slab_rows = half + LANE
    trash = half
    stripe = half // NS
    zr = stripe // 4

    assert share % sb == 0 and stripe % 8 == 0 and cap % G == 0 and zr % 8 == 0

    mesh = plsc.VectorSubcoreMesh(core_axis_name="c", subcore_axis_name="s")

    if want_cnt:
        out_type = [jax.ShapeDtypeStruct((NUM_REL, npad, D), jnp.float32),
                    jax.ShapeDtypeStruct((NUM_REL, npad, D), jnp.float32)]
    else:
        out_type = jax.ShapeDtypeStruct((NUM_REL, npad, D), jnp.float32)

    scratch = [
        pltpu.VMEM((2, sb), jnp.int32),
        pltpu.VMEM((2, sb), jnp.int32),
        pltpu.VMEM((2, sb), jnp.int32),
        pltpu.VMEM((cap,), jnp.int32),
        pltpu.VMEM((cap,), jnp.int32),
        pltpu.VMEM((G, D), jnp.float32),
        pltpu.VMEM((G,), jnp.int32),
        pltpu.VMEM((G,), jnp.int32),
        pltpu.VMEM((zr, D), jnp.float32),
        pltpu.VMEM_SHARED((slab_rows, D), jnp.float32),
        pltpu.SemaphoreType.DMA,
        pltpu.SemaphoreType.DMA,
        pltpu.SemaphoreType.DMA,
    ]
    if want_cnt:
        scratch[9:9] = [pltpu.VMEM((G, D), jnp.float32)]

    @functools.partial(
        pl.kernel, out_type=out_type, mesh=mesh, scratch_types=scratch,
        compiler_params=pltpu.CompilerParams(needs_layout_passes=False))
    def agg_kernel(x_hbm, src_hbm, dst_hbm, typ_hbm, zrow_hbm, ones_hbm,
                   agg_hbm, *rest):
        if want_cnt:
            cnt_hbm = rest[0]
            (ebuf_s, ebuf_d, ebuf_t, match_src, match_dst, rows, idx_gather,
             idx_stage, zbuf, ones_v, slab, esem0, esem1, gsem) = rest[1:]
        else:
            (ebuf_s, ebuf_d, ebuf_t, match_src, match_dst, rows, idx_gather,
             idx_stage, zbuf, slab, esem0, esem1, gsem) = rest
            cnt_hbm = ones_v = None

        c = lax.axis_index("c")
        s = lax.axis_index("s")
        s_base = s * share
        row0 = s * stripe

        pltpu.sync_copy(zrow_hbm, zbuf)
        if want_cnt:
            pltpu.sync_copy(ones_hbm, ones_v)

        esems = (esem0, esem1)

        def fire(b, slot):
            base = s_base + b * sb
            sem = esems[slot]
            return (
                pltpu.async_copy(src_hbm.at[pl.ds(base, sb)], ebuf_s.at[slot], sem),
                pltpu.async_copy(dst_hbm.at[pl.ds(base, sb)], ebuf_d.at[slot], sem),
                pltpu.async_copy(typ_hbm.at[pl.ds(base, sb)], ebuf_t.at[slot], sem),
            )

        def zero_stripe():
            for z in range(stripe // zr):
                pltpu.sync_copy(zbuf, slab.at[pl.ds(row0 + z * zr, zr)])

        def copy_stripe(dst_ref, r_idx, hbase):
            pltpu.sync_copy(slab.at[pl.ds(row0, stripe)],
                            dst_ref.at[r_idx, pl.ds(hbase + row0, stripe)])

        pad_src = jnp.zeros((LANE,), jnp.int32)
        pad_dst = jnp.full((LANE,), trash, jnp.int32)

        def pass_body(p, pcarry):
            r_idx = c * rel_per_core + p // nq
            hbase = (p % nq) * half

            zero_stripe()
            plsc.subcore_barrier()

            def make_sbody(slot):
                def sbody(i, m):
                    off = i * LANE
                    sv = ebuf_s[slot, pl.ds(off, LANE)]
                    dv = ebuf_d[slot, pl.ds(off, LANE)]
                    tv = ebuf_t[slot, pl.ds(off, LANE)]
                    dl = dv - hbase
                    mask = (tv == r_idx) & (dl >= 0) & (dl < half)
                    ones_m = mask.astype(jnp.int32)
                    inc = jnp.cumsum(ones_m)
                    pos = jnp.where(mask, m + inc - 1,
                                    cap - LANE + lax.iota(jnp.int32, LANE))
                    plsc.store_scatter(match_src, [pos], sv)
                    plsc.store_scatter(match_dst, [pos], dl)
                    return m + jnp.sum(ones_m)
                return sbody

            descs = {0: fire(0, 0)}
            m = jnp.int32(0)
            for b in range(nb):
                slot = b % 2
                if b + 1 < nb:
                    descs[b + 1] = fire(b + 1, 1 - slot)
                for dd in descs.pop(b):
                    dd.wait()
                m = lax.fori_loop(0, sb // LANE, make_sbody(slot), m)

            for t in range(G // LANE):
                match_src[pl.ds(m + t * LANE, LANE)] = pad_src
                match_dst[pl.ds(m + t * LANE, LANE)] = pad_dst

            def gbody(g, carry):
                @pl.when(g * G < m)
                def _():
                    for t in range(G // LANE):
                        idx_gather[pl.ds(t * LANE, LANE)] = (
                            match_src[pl.ds(g * G + t * LANE, LANE)])
                        idx_stage[pl.ds(t * LANE, LANE)] = (
                            match_dst[pl.ds(g * G + t * LANE, LANE)])
                    pltpu.async_copy(x_hbm.at[idx_gather], rows, gsem).wait()
                    pltpu.sync_copy(rows, slab.at[idx_stage], add=True)
                return carry
            lax.fori_loop(0, ng, gbody, jnp.int32(0))

            plsc.subcore_barrier()
            copy_stripe(agg_hbm, r_idx, hbase)

            if want_cnt:
                zero_stripe()
                plsc.subcore_barrier()

                def cbody(g, carry):
                    @pl.when(g * G < m)
                    def _():
                        for t in range(G // LANE):
                            idx_stage[pl.ds(t * LANE, LANE)] = (
                                match_dst[pl.ds(g * G + t * LANE, LANE)])
                        pltpu.sync_copy(ones_v, slab.at[idx_stage], add=True)
                    return carry
                lax.fori_loop(0, ng, cbody, jnp.int32(0))

                plsc.subcore_barrier()
                copy_stripe(cnt_hbm, r_idx, hbase)

            return pcarry

        lax.fori_loop(0, rel_per_core * nq, pass_body, jnp.int32(0))

    return agg_kernel


def _dense_body(relu, x_ref, agg_ref, cnt_ref, w_ref, root_ref, b_ref, o_ref):
    acc = jnp.dot(x_ref[...], root_ref[...],
                  preferred_element_type=jnp.float32) + b_ref[...]
    for rr in range(NUM_REL):
        a = agg_ref[rr]
        cnt = cnt_ref[rr, :, 0:1]
        a = a / jnp.maximum(cnt, 1.0)
        acc = acc + jnp.dot(a, w_ref[rr], preferred_element_type=jnp.float32)
    if relu:
        acc = jnp.maximum(acc, 0.0)
    o_ref[...] = acc


def _dense(x, agg, cnt, W, root, b, relu):
    N, D = x.shape
    blk = 256
    nblk = (N + blk - 1) // blk
    return pl.pallas_call(
        functools.partial(_dense_body, relu),
        grid=(nblk,),
        in_specs=[
            pl.BlockSpec((blk, D), lambda i: (i, 0)),
            pl.BlockSpec((NUM_REL, blk, D), lambda i: (0, i, 0)),
            pl.BlockSpec((NUM_REL, blk, D), lambda i: (0, i, 0)),
            pl.BlockSpec((NUM_REL, D, D), lambda i: (0, 0, 0)),
            pl.BlockSpec((D, D), lambda i: (0, 0)),
            pl.BlockSpec((1, D), lambda i: (0, 0)),
        ],
        out_specs=pl.BlockSpec((blk, D), lambda i: (i, 0)),
        out_shape=jax.ShapeDtypeStruct((N, D), jnp.float32),
    )(x, agg, cnt, W, root, b.reshape(1, D))


def kernel(x, edge_indexes, edge_types, W1, root1, b1, W2, root2, b2):
    N, D = x.shape
    E = edge_types.shape[0]
    src = edge_indexes[0]
    dst = edge_indexes[1]

    half = -(-(-(-N // 4)) // (8 * NS)) * (8 * NS)
    zr = (half // NS) // 4
    zrow = jnp.zeros((zr, D), jnp.float32)
    ones = jnp.ones((G, D), jnp.float32)

    sb = 2048
    share = -(-(-(-E // NS)) // sb) * sb
    e_pad = NS * share
    pad_n = e_pad - E
    src = jnp.pad(src, (0, pad_n))
    dst = jnp.pad(dst, (0, pad_n))
    typ = jnp.pad(edge_types, (0, pad_n), constant_values=-1)

    agg_cnt = _make_agg(N, e_pad, D, True)
    agg_only = _make_agg(N, e_pad, D, False)

    agg1, cnt = agg_cnt(x, src, dst, typ, zrow, ones)
    h = _dense(x, agg1, cnt, W1, root1, b1, True)
    agg2 = agg_only(h, src, dst, typ, zrow, ones)
    out = _dense(h, agg2, cnt, W2, root2, b2, False)
    return out
